# baseline (speedup 1.0000x reference)
import jax
import jax.numpy as jnp
from jax import lax
from jax.experimental import pallas as pl
from jax.experimental.pallas import tpu as pltpu
from jax.experimental.pallas import tpu_sc as plsc

_B, _S = 1024, 20
_N = _B * _S
_D = 4096
_NC, _NS = 2, 16
_NW = _NC * _NS
_BPW = _N // _NW
_CA = 16
_CB = 8


def kernel(indices, weight):
    idx_smajor = indices.T.reshape(_N).astype(jnp.int32)

    mesh = plsc.VectorSubcoreMesh(
        core_axis_name="core", subcore_axis_name="subcore"
    )

    @pl.kernel(
        out_type=jax.ShapeDtypeStruct((_N, _D), jnp.float32),
        mesh=mesh,
        scratch_types=[
            pltpu.VMEM((_BPW,), jnp.int32),
            pltpu.VMEM((_CA, _D), jnp.float32),
            pltpu.VMEM((_CB, _D), jnp.float32),
            pltpu.SemaphoreType.DMA,
            pltpu.SemaphoreType.DMA,
        ],
    )
    def gather_kernel(w_hbm, i_hbm, o_hbm, idx_v, buf_a, buf_b, sem_a, sem_b):
        wid = lax.axis_index("subcore") * _NC + lax.axis_index("core")
        base = wid * _BPW
        pltpu.sync_copy(i_hbm.at[pl.ds(base, _BPW)], idx_v)

        def start_gather(c, buf, sem, n):
            pltpu.make_async_copy(
                w_hbm.at[idx_v.at[pl.ds(c, n)]], buf, sem
            ).start()

        def wait_gather(buf, sem, n):
            pltpu.make_async_copy(w_hbm.at[idx_v.at[pl.ds(0, n)]], buf, sem).wait()

        start_gather(0, buf_a, sem_a, _CA)
        start_gather(_CA, buf_b, sem_b, _CB)
        _STEP = _CA + _CB

        @pl.loop(0, _BPW - _CA, step=_STEP)
        def _(c):
            wait_gather(buf_a, sem_a, _CA)
            pltpu.sync_copy(buf_a, o_hbm.at[pl.ds(base + c, _CA)])
            start_gather(c + _STEP, buf_a, sem_a, _CA)

            wait_gather(buf_b, sem_b, _CB)
            pltpu.sync_copy(buf_b, o_hbm.at[pl.ds(base + c + _CA, _CB)])

            @pl.when(c + _STEP + _CA + _CB <= _BPW)
            def _():
                start_gather(c + _STEP + _CA, buf_b, sem_b, _CB)

        wait_gather(buf_a, sem_a, _CA)
        pltpu.sync_copy(buf_a, o_hbm.at[pl.ds(base + _BPW - _CA, _CA)])

    out_flat = gather_kernel(weight, idx_smajor)
    return jnp.transpose(out_flat.reshape(_S, _B, _D), (1, 0, 2))


# speedup vs baseline: 1.0157x; 1.0157x over previous
import jax
import jax.numpy as jnp
from jax import lax
from jax.experimental import pallas as pl
from jax.experimental.pallas import tpu as pltpu
from jax.experimental.pallas import tpu_sc as plsc

_B, _S = 1024, 20
_N = _B * _S
_D = 4096
_NC, _NS = 2, 16
_NW = _NC * _NS
_BPW = _N // _NW
_C = 8


def kernel(indices, weight):
    idx_smajor = indices.T.reshape(_N).astype(jnp.int32)

    mesh = plsc.VectorSubcoreMesh(
        core_axis_name="core", subcore_axis_name="subcore"
    )

    @pl.kernel(
        out_type=jax.ShapeDtypeStruct((_N, _D), jnp.float32),
        mesh=mesh,
        scratch_types=[
            pltpu.VMEM((_BPW,), jnp.int32),
            pltpu.VMEM((_C, _D), jnp.float32),
            pltpu.VMEM((_C, _D), jnp.float32),
            pltpu.VMEM((_C, _D), jnp.float32),
            pltpu.SemaphoreType.DMA,
            pltpu.SemaphoreType.DMA,
            pltpu.SemaphoreType.DMA,
        ],
    )
    def gather_kernel(w_hbm, i_hbm, o_hbm, idx_v, buf0, buf1, buf2,
                      sem0, sem1, sem2):
        wid = lax.axis_index("subcore") * _NC + lax.axis_index("core")
        base = wid * _BPW
        pltpu.sync_copy(i_hbm.at[pl.ds(base, _BPW)], idx_v)

        def start_gather(c, buf, sem):
            pltpu.make_async_copy(
                w_hbm.at[idx_v.at[pl.ds(c, _C)]], buf, sem
            ).start()

        def wait_gather(buf, sem):
            pltpu.make_async_copy(w_hbm.at[idx_v.at[pl.ds(0, _C)]], buf, sem).wait()

        bufs = (buf0, buf1, buf2)
        sems = (sem0, sem1, sem2)
        for k in range(3):
            start_gather(k * _C, bufs[k], sems[k])

        @pl.loop(0, _BPW - 2 * _C, step=3 * _C)
        def _(c):
            for k in range(3):
                wait_gather(bufs[k], sems[k])
                pltpu.sync_copy(
                    bufs[k], o_hbm.at[pl.ds(base + c + k * _C, _C)]
                )

                @pl.when(c + (k + 3) * _C + _C <= _BPW)
                def _():
                    start_gather(c + (k + 3) * _C, bufs[k], sems[k])

        for k in range(2):
            wait_gather(bufs[k], sems[k])
            pltpu.sync_copy(
                bufs[k], o_hbm.at[pl.ds(base + _BPW - (2 - k) * _C, _C)]
            )

    out_flat = gather_kernel(weight, idx_smajor)
    return jnp.transpose(out_flat.reshape(_S, _B, _D), (1, 0, 2))
